# baseline (speedup 1.0000x reference)
import functools

import jax
import jax.numpy as jnp
from jax import lax
from jax.experimental import pallas as pl
from jax.experimental.pallas import tpu as pltpu
from jax.experimental.pallas import tpu_sc as plsc

NC = 2
NS = 16
NW = NC * NS
LANES = 16


def _sc_lookup_add(idx, tv, emb_pk, *, chunk):
    n, d = tv.shape
    nh, hlen = emb_pk.shape
    dp = d // 2 // NC
    dh = d // NC
    n_per_w = n // NS
    n_chunks = n_per_w // chunk
    assert n_chunks % 2 == 0 and chunk % LANES == 0
    mesh = plsc.VectorSubcoreMesh(core_axis_name="c", subcore_axis_name="s")

    @functools.partial(
        pl.kernel,
        mesh=mesh,
        compiler_params=pltpu.CompilerParams(
            use_tc_tiling_on_sc=False, needs_layout_passes=False
        ),
        out_type=jax.ShapeDtypeStruct((n, d), jnp.float32),
        scratch_types=[
            pltpu.VMEM((hlen,), jnp.int32),
            pltpu.VMEM((chunk,), jnp.int32),
            pltpu.VMEM((chunk,), jnp.int32),
            pltpu.VMEM((chunk, dh), jnp.float32),
            pltpu.VMEM((chunk, dh), jnp.float32),
            pltpu.SemaphoreType.DMA,
            pltpu.SemaphoreType.DMA,
            pltpu.SemaphoreType.DMA,
            pltpu.SemaphoreType.DMA,
            pltpu.SemaphoreType.DMA,
            pltpu.SemaphoreType.DMA,
        ],
    )
    def k(idx_hbm, tv_hbm, emb_hbm, out_hbm, emb_t,
          ix0, ix1, tv0, tv1, st0, st1, si0, si1, so0, so1):
        half = lax.axis_index("c")
        rbase = lax.axis_index("s") * n_per_w
        col0 = half * dh
        tv_bufs = (tv0, tv1)
        ix_bufs = (ix0, ix1)
        sem_tv = (st0, st1)
        sem_ix = (si0, si1)
        sem_out = (so0, so1)

        pltpu.sync_copy(emb_hbm.at[half], emb_t)

        def start_in(c, b):
            row0 = c * chunk
            pltpu.async_copy(
                tv_hbm.at[pl.ds(rbase + row0, chunk), pl.ds(col0, dh)],
                tv_bufs[b], sem_tv[b],
            )
            pltpu.async_copy(
                idx_hbm.at[pl.ds(rbase + row0, chunk)], ix_bufs[b], sem_ix[b]
            )

        def wait_in(b):
            pltpu.make_async_copy(
                tv_hbm.at[pl.ds(rbase, chunk), pl.ds(col0, dh)],
                tv_bufs[b], sem_tv[b],
            ).wait()
            pltpu.make_async_copy(
                idx_hbm.at[pl.ds(rbase, chunk)], ix_bufs[b], sem_ix[b]
            ).wait()

        def wait_out(b):
            pltpu.make_async_copy(
                tv_bufs[b], out_hbm.at[pl.ds(rbase, chunk), pl.ds(col0, dh)],
                sem_out[b],
            ).wait()

        lanes = lax.iota(jnp.int32, LANES)

        def add_chunk(b):
            def group_body(g, carry):
                iv = ix_bufs[b][pl.ds(g * LANES, LANES)]
                rows = g * LANES + lanes

                def col_body(c, carry2):
                    pk = plsc.load_gather(emb_t, [iv * dp + c])
                    lo, hi = plsc.unpack(
                        plsc.bitcast(pk, jnp.bfloat16),
                        format=plsc.PackFormat.INTERLEAVED,
                    )
                    cc = jnp.full((LANES,), 2 * c, jnp.int32)
                    plsc.addupdate_scatter(tv_bufs[b], [rows, cc], lo)
                    plsc.addupdate_scatter(tv_bufs[b], [rows, cc + 1], hi)
                    return carry2

                return lax.fori_loop(0, dp, col_body, carry, unroll=4)

            lax.fori_loop(0, chunk // LANES, group_body, 0)

        start_in(0, 0)

        def pair_body(i, carry):
            for b in (0, 1):
                c = 2 * i + b
                q = 1 - b
                if b == 0:
                    @pl.when(i > 0)
                    def _():
                        wait_out(q)
                    start_in(c + 1, q)
                else:
                    wait_out(q)

                    @pl.when(i < n_chunks // 2 - 1)
                    def _():
                        start_in(c + 1, q)
                wait_in(b)
                add_chunk(b)
                row0 = c * chunk
                pltpu.async_copy(
                    tv_bufs[b],
                    out_hbm.at[pl.ds(rbase + row0, chunk), pl.ds(col0, dh)],
                    sem_out[b],
                )
            return carry

        lax.fori_loop(0, n_chunks // 2, pair_body, 0)
        wait_out(1)

    return k(idx, tv, emb_pk)


def kernel(prefix, time_vector, embedding):
    b, s = prefix.shape
    v, d = embedding.shape
    n = b * s
    idx = prefix.reshape(n).astype(jnp.int32)
    tv = time_vector.reshape(n, d)
    packed = jax.lax.bitcast_convert_type(
        embedding.astype(jnp.bfloat16).reshape(v, d // 2, 2), jnp.int32
    )
    dp = d // 2 // NC
    emb_pk = packed.reshape(v, NC, dp).transpose(1, 0, 2).reshape(NC, v * dp)
    out = _sc_lookup_add(idx, tv, emb_pk, chunk=64)
    return out.reshape(b, s, d)


# speedup vs baseline: 3.9033x; 3.9033x over previous
import functools

import jax
import jax.numpy as jnp
from jax import lax
from jax.experimental import pallas as pl
from jax.experimental.pallas import tpu as pltpu
from jax.experimental.pallas import tpu_sc as plsc

NC = 2
NS = 16
NW = NC * NS
LANES = 16


def _sc_lookup_add(idx, tv, emb_pk, *, chunk):
    n, d = tv.shape
    nh, hlen = emb_pk.shape
    dp = d // 2 // NC
    dh = d // NC
    n_per_w = n // NS
    n_chunks = n_per_w // chunk
    assert n_chunks % 2 == 0 and chunk % LANES == 0
    mesh = plsc.VectorSubcoreMesh(core_axis_name="c", subcore_axis_name="s")

    @functools.partial(
        pl.kernel,
        mesh=mesh,
        compiler_params=pltpu.CompilerParams(
            use_tc_tiling_on_sc=False, needs_layout_passes=False
        ),
        out_type=jax.ShapeDtypeStruct((n, d), jnp.float32),
        scratch_types=[
            pltpu.VMEM((hlen,), jnp.int32),
            pltpu.VMEM((chunk,), jnp.int32),
            pltpu.VMEM((chunk,), jnp.int32),
            pltpu.VMEM((chunk, dh), jnp.float32),
            pltpu.VMEM((chunk, dh), jnp.float32),
            pltpu.SemaphoreType.DMA,
            pltpu.SemaphoreType.DMA,
            pltpu.SemaphoreType.DMA,
            pltpu.SemaphoreType.DMA,
            pltpu.SemaphoreType.DMA,
            pltpu.SemaphoreType.DMA,
        ],
    )
    def k(idx_hbm, tv_hbm, emb_hbm, out_hbm, emb_t,
          ix0, ix1, tv0, tv1, st0, st1, si0, si1, so0, so1):
        half = lax.axis_index("c")
        rbase = lax.axis_index("s") * n_per_w
        col0 = half * dh
        tv_bufs = (tv0, tv1)
        ix_bufs = (ix0, ix1)
        sem_tv = (st0, st1)
        sem_ix = (si0, si1)
        sem_out = (so0, so1)

        pltpu.sync_copy(emb_hbm.at[half], emb_t)

        def start_in(c, b):
            row0 = c * chunk
            pltpu.async_copy(
                tv_hbm.at[pl.ds(rbase + row0, chunk), pl.ds(col0, dh)],
                tv_bufs[b], sem_tv[b],
            )
            pltpu.async_copy(
                idx_hbm.at[pl.ds(rbase + row0, chunk)], ix_bufs[b], sem_ix[b]
            )

        def wait_in(b):
            pltpu.make_async_copy(
                tv_hbm.at[pl.ds(rbase, chunk), pl.ds(col0, dh)],
                tv_bufs[b], sem_tv[b],
            ).wait()
            pltpu.make_async_copy(
                idx_hbm.at[pl.ds(rbase, chunk)], ix_bufs[b], sem_ix[b]
            ).wait()

        def wait_out(b):
            pltpu.make_async_copy(
                tv_bufs[b], out_hbm.at[pl.ds(rbase, chunk), pl.ds(col0, dh)],
                sem_out[b],
            ).wait()

        lanes = lax.iota(jnp.int32, LANES)

        def add_chunk(b):
            def group_body(g, carry):
                iv = ix_bufs[b][pl.ds(g * LANES, LANES)]
                rows = g * LANES + lanes
                ivdp = iv * dp

                @functools.partial(plsc.parallel_loop, 0, dp, unroll=8)
                def col_body(c):
                    pk = plsc.load_gather(emb_t, [ivdp + c])
                    lo, hi = plsc.unpack(
                        plsc.bitcast(pk, jnp.bfloat16),
                        format=plsc.PackFormat.INTERLEAVED,
                    )
                    cc = jnp.full((LANES,), 2 * c, jnp.int32)
                    plsc.addupdate_scatter(tv_bufs[b], [rows, cc], lo)
                    plsc.addupdate_scatter(tv_bufs[b], [rows, cc + 1], hi)

                return carry

            lax.fori_loop(0, chunk // LANES, group_body, 0)

        start_in(0, 0)

        def pair_body(i, carry):
            for b in (0, 1):
                c = 2 * i + b
                q = 1 - b
                if b == 0:
                    @pl.when(i > 0)
                    def _():
                        wait_out(q)
                    start_in(c + 1, q)
                else:
                    wait_out(q)

                    @pl.when(i < n_chunks // 2 - 1)
                    def _():
                        start_in(c + 1, q)
                wait_in(b)
                add_chunk(b)
                row0 = c * chunk
                pltpu.async_copy(
                    tv_bufs[b],
                    out_hbm.at[pl.ds(rbase + row0, chunk), pl.ds(col0, dh)],
                    sem_out[b],
                )
            return carry

        lax.fori_loop(0, n_chunks // 2, pair_body, 0)
        wait_out(1)

    return k(idx, tv, emb_pk)


def kernel(prefix, time_vector, embedding):
    b, s = prefix.shape
    v, d = embedding.shape
    n = b * s
    idx = prefix.reshape(n).astype(jnp.int32)
    tv = time_vector.reshape(n, d)
    packed = jax.lax.bitcast_convert_type(
        embedding.astype(jnp.bfloat16).reshape(v, d // 2, 2), jnp.int32
    )
    dp = d // 2 // NC
    emb_pk = packed.reshape(v, NC, dp).transpose(1, 0, 2).reshape(NC, v * dp)
    out = _sc_lookup_add(idx, tv, emb_pk, chunk=64)
    return out.reshape(b, s, d)
